# Initial kernel scaffold; baseline (speedup 1.0000x reference)
import jax
import jax.numpy as jnp
from jax.experimental import pallas as pl


def kernel(x, pe, edge_index, batch, atom_emb, pe_W1, pe_b1, pe_W2, pe_b2, pe_bn_g, pe_bn_b, in_W, in_b, gin_W1, gin_b1, gin_W2, gin_b2, bn_g, bn_b, ro_W1, ro_b1, ro_W2, ro_b2):
    raise NotImplementedError("write your pallas kernel here")


# speedup vs baseline: 4.1653x; 4.1653x over previous
import functools

import jax
import jax.numpy as jnp
from jax import lax
from jax.experimental import pallas as pl
from jax.experimental.pallas import tpu as pltpu
from jax.experimental.pallas import tpu_sc as plsc

N = 50000
E = 800000
H = 64
HH = 32
A = 64
P = 16
PR = 20
NA = 28
G = 2000
L = 4

NC = 2
NS = 16

ELANES = 128
CHUNK_ROWS = 4
NCHUNKS = 100
EPS_ROWS = CHUNK_ROWS * NCHUNKS
ROWS_TOT = EPS_ROWS * NS
E_PAD = ROWS_TOT * ELANES
CPY = 1000
NCPY = N // CPY

BLK = 2000
GRID = N // BLK

@functools.cache
def _mesh():
    return plsc.VectorSubcoreMesh(core_axis_name="c", subcore_axis_name="s",
                                  num_cores=NC, num_subcores=NS)


def _agg_body(h_hbm, srcr_hbm, dstr_hbm, z_hbm, acc, idx_s, idx_d, rows):
    c = lax.axis_index("c")
    s = lax.axis_index("s")
    for k in range(4):
        cid = s + NS * k

        @pl.when(cid < NCPY)
        def _():
            pltpu.sync_copy(h_hbm.at[c, pl.ds(cid * CPY, CPY)],
                            acc.at[pl.ds(cid * CPY, CPY)])
    plsc.subcore_barrier()
    row0 = s * EPS_ROWS

    def chunk(k, carry):
        r0 = row0 + k * CHUNK_ROWS
        pltpu.sync_copy(srcr_hbm.at[pl.ds(r0, CHUNK_ROWS)], idx_s)
        pltpu.sync_copy(dstr_hbm.at[pl.ds(r0, CHUNK_ROWS)], idx_d)
        for j in range(CHUNK_ROWS):
            pltpu.sync_copy(h_hbm.at[c].at[idx_s.at[j]], rows.at[j])
            pltpu.sync_copy(rows.at[j], acc.at[idx_d.at[j]], add=True)
        return carry

    lax.fori_loop(0, NCHUNKS, chunk, 0)
    plsc.subcore_barrier()
    for k in range(4):
        cid = s + NS * k

        @pl.when(cid < NCPY)
        def _():
            pltpu.sync_copy(acc.at[pl.ds(cid * CPY, CPY)],
                            z_hbm.at[c, pl.ds(cid * CPY, CPY)])


@functools.cache
def _agg_kernel():
    return pl.kernel(
        _agg_body,
        out_type=jax.ShapeDtypeStruct((NC, N, HH), jnp.float32),
        mesh=_mesh(),
        compiler_params=pltpu.CompilerParams(use_tc_tiling_on_sc=False),
        scratch_types=[
            pltpu.VMEM_SHARED((N + NS, HH), jnp.float32),
            pltpu.VMEM((CHUNK_ROWS, ELANES), jnp.int32),
            pltpu.VMEM((CHUNK_ROWS, ELANES), jnp.int32),
            pltpu.VMEM((CHUNK_ROWS, ELANES, HH), jnp.float32),
        ],
    )


_PCHUNK = 8
_PNCH = (N // 125) // _PCHUNK
_GCPY = 200


def _pool_body(hp_hbm, batchr_hbm, zeros_hbm, out_hbm, acc, idx_b, rows):
    c = lax.axis_index("c")
    s = lax.axis_index("s")

    @pl.when(s < G // _GCPY)
    def _():
        pltpu.sync_copy(zeros_hbm.at[pl.ds(s * _GCPY, _GCPY)],
                        acc.at[pl.ds(s * _GCPY, _GCPY)])
    plsc.subcore_barrier()

    for k in range(4):
        cid = s + NS * k

        @pl.when(cid < _PNCH)
        def _():
            pltpu.sync_copy(batchr_hbm.at[pl.ds(cid * _PCHUNK, _PCHUNK)], idx_b)
            pltpu.sync_copy(hp_hbm.at[c, pl.ds(cid * _PCHUNK, _PCHUNK)], rows)
            for j in range(_PCHUNK):
                pltpu.sync_copy(rows.at[j], acc.at[idx_b.at[j]], add=True)

    plsc.subcore_barrier()

    @pl.when(s < G // _GCPY)
    def _():
        pltpu.sync_copy(acc.at[pl.ds(s * _GCPY, _GCPY)],
                        out_hbm.at[c, pl.ds(s * _GCPY, _GCPY)])


@functools.cache
def _pool_kernel():
    return pl.kernel(
        _pool_body,
        out_type=jax.ShapeDtypeStruct((NC, G, HH), jnp.float32),
        mesh=_mesh(),
        compiler_params=pltpu.CompilerParams(use_tc_tiling_on_sc=False),
        scratch_types=[
            pltpu.VMEM_SHARED((G, HH), jnp.float32),
            pltpu.VMEM((_PCHUNK, 125), jnp.int32),
            pltpu.VMEM((_PCHUNK, 125, HH), jnp.float32),
        ],
    )


def _pe_body(pe_ref, w1_ref, b1_ref, w2_ref, b2_ref, xp_ref, st_ref):
    i = pl.program_id(0)
    y = jnp.maximum(
        jnp.dot(pe_ref[...], w1_ref[...], preferred_element_type=jnp.float32,
                precision=lax.Precision.HIGHEST)
        + b1_ref[...], 0.0)
    z = jnp.dot(y, w2_ref[...], preferred_element_type=jnp.float32,
                precision=lax.Precision.HIGHEST) + b2_ref[...]
    xp_ref[...] = z

    @pl.when(i == 0)
    def _():
        st_ref[...] = jnp.zeros_like(st_ref)

    st_ref[0:1, :] += jnp.sum(z, axis=0, keepdims=True)
    st_ref[1:2, :] += jnp.sum(z * z, axis=0, keepdims=True)


def _pe_pass(pe, w1, b1, w2, b2):
    return pl.pallas_call(
        _pe_body,
        grid=(GRID,),
        in_specs=[
            pl.BlockSpec((BLK, PR), lambda i: (i, 0)),
            pl.BlockSpec((PR, P), lambda i: (0, 0)),
            pl.BlockSpec((1, P), lambda i: (0, 0)),
            pl.BlockSpec((P, P), lambda i: (0, 0)),
            pl.BlockSpec((1, P), lambda i: (0, 0)),
        ],
        out_specs=[
            pl.BlockSpec((BLK, P), lambda i: (i, 0)),
            pl.BlockSpec((2, P), lambda i: (0, 0)),
        ],
        out_shape=[
            jax.ShapeDtypeStruct((N, P), jnp.float32),
            jax.ShapeDtypeStruct((2, P), jnp.float32),
        ],
    )(pe, w1, b1, w2, b2)


def _h0_body(x_ref, xp_ref, st_ref, aemb_ref, wa_ref, wp_ref, inb_ref,
             peg_ref, peb_ref, out_ref):
    mu = st_ref[0:1, :] * (1.0 / N)
    var = st_ref[1:2, :] * (1.0 / N) - mu * mu
    scale = peg_ref[...] * lax.rsqrt(var + 1e-5)
    shift = peb_ref[...] - mu * scale
    onehot = (x_ref[...] == lax.broadcasted_iota(jnp.int32, (1, NA), 1)
              ).astype(jnp.float32)
    m = jnp.dot(aemb_ref[...], wa_ref[...], preferred_element_type=jnp.float32,
                precision=lax.Precision.HIGHEST)
    xa_part = jnp.dot(onehot, m, preferred_element_type=jnp.float32,
                precision=lax.Precision.HIGHEST)
    xp_bn = xp_ref[...] * scale + shift
    h0 = (xa_part
          + jnp.dot(xp_bn, wp_ref[...], preferred_element_type=jnp.float32,
                precision=lax.Precision.HIGHEST)
          + inb_ref[...])
    out_ref[0] = h0[:, :HH]
    out_ref[1] = h0[:, HH:]


def _h0_pass(x2d, xp, st, aemb, wa, wp, inb, peg, peb):
    return pl.pallas_call(
        _h0_body,
        grid=(GRID,),
        in_specs=[
            pl.BlockSpec((BLK, 1), lambda i: (i, 0)),
            pl.BlockSpec((BLK, P), lambda i: (i, 0)),
            pl.BlockSpec((2, P), lambda i: (0, 0)),
            pl.BlockSpec((NA, A), lambda i: (0, 0)),
            pl.BlockSpec((A, H), lambda i: (0, 0)),
            pl.BlockSpec((P, H), lambda i: (0, 0)),
            pl.BlockSpec((1, H), lambda i: (0, 0)),
            pl.BlockSpec((1, P), lambda i: (0, 0)),
            pl.BlockSpec((1, P), lambda i: (0, 0)),
        ],
        out_specs=pl.BlockSpec((NC, BLK, HH), lambda i: (0, i, 0)),
        out_shape=jax.ShapeDtypeStruct((NC, N, HH), jnp.float32),
    )(x2d, xp, st, aemb, wa, wp, inb, peg, peb)


def _mlp_body(z_ref, w1_ref, b1_ref, w2_ref, b2_ref, z2_ref, st_ref):
    i = pl.program_id(0)
    z = jnp.concatenate([z_ref[0], z_ref[1]], axis=1)
    y = jnp.maximum(
        jnp.dot(z, w1_ref[...], preferred_element_type=jnp.float32,
                precision=lax.Precision.HIGHEST)
        + b1_ref[...], 0.0)
    z2 = jnp.dot(y, w2_ref[...], preferred_element_type=jnp.float32,
                precision=lax.Precision.HIGHEST) + b2_ref[...]
    z2_ref[...] = z2

    @pl.when(i == 0)
    def _():
        st_ref[...] = jnp.zeros_like(st_ref)

    st_ref[0:1, :] += jnp.sum(z2, axis=0, keepdims=True)
    st_ref[1:2, :] += jnp.sum(z2 * z2, axis=0, keepdims=True)


def _mlp_pass(z_halves, w1, b1, w2, b2):
    return pl.pallas_call(
        _mlp_body,
        grid=(GRID,),
        in_specs=[
            pl.BlockSpec((NC, BLK, HH), lambda i: (0, i, 0)),
            pl.BlockSpec((H, H), lambda i: (0, 0)),
            pl.BlockSpec((1, H), lambda i: (0, 0)),
            pl.BlockSpec((H, H), lambda i: (0, 0)),
            pl.BlockSpec((1, H), lambda i: (0, 0)),
        ],
        out_specs=[
            pl.BlockSpec((BLK, H), lambda i: (i, 0)),
            pl.BlockSpec((2, H), lambda i: (0, 0)),
        ],
        out_shape=[
            jax.ShapeDtypeStruct((N, H), jnp.float32),
            jax.ShapeDtypeStruct((2, H), jnp.float32),
        ],
    )(z_halves, w1, b1, w2, b2)


def _bnrelu_body(z2_ref, st_ref, g_ref, b_ref, out_ref):
    mu = st_ref[0:1, :] * (1.0 / N)
    var = st_ref[1:2, :] * (1.0 / N) - mu * mu
    scale = g_ref[...] * lax.rsqrt(var + 1e-5)
    shift = b_ref[...] - mu * scale
    hh = jnp.maximum(z2_ref[...] * scale + shift, 0.0)
    out_ref[0] = hh[:, :HH]
    out_ref[1] = hh[:, HH:]


def _bnrelu_pass(z2, st, g, b):
    return pl.pallas_call(
        _bnrelu_body,
        grid=(GRID,),
        in_specs=[
            pl.BlockSpec((BLK, H), lambda i: (i, 0)),
            pl.BlockSpec((2, H), lambda i: (0, 0)),
            pl.BlockSpec((1, H), lambda i: (0, 0)),
            pl.BlockSpec((1, H), lambda i: (0, 0)),
        ],
        out_specs=pl.BlockSpec((NC, BLK, HH), lambda i: (0, i, 0)),
        out_shape=jax.ShapeDtypeStruct((NC, N, HH), jnp.float32),
    )(z2, st, g, b)


def _ro_body(p_ref, w1_ref, b1_ref, w2_ref, b2_ref, out_ref):
    p = jnp.concatenate([p_ref[0], p_ref[1]], axis=1)
    r = jnp.maximum(
        jnp.dot(p, w1_ref[...], preferred_element_type=jnp.float32,
                precision=lax.Precision.HIGHEST)
        + b1_ref[...], 0.0)
    out_ref[...] = (jnp.dot(r, w2_ref[...], preferred_element_type=jnp.float32,
                precision=lax.Precision.HIGHEST)
                    + b2_ref[...])


def _ro_pass(pooled, w1, b1, w2, b2):
    return pl.pallas_call(
        _ro_body,
        in_specs=[
            pl.BlockSpec((NC, G, HH), lambda: (0, 0, 0)),
            pl.BlockSpec((H, H), lambda: (0, 0)),
            pl.BlockSpec((1, H), lambda: (0, 0)),
            pl.BlockSpec((H, 1), lambda: (0, 0)),
            pl.BlockSpec((1, 1), lambda: (0, 0)),
        ],
        out_specs=pl.BlockSpec((G, 1), lambda: (0, 0)),
        out_shape=jax.ShapeDtypeStruct((G, 1), jnp.float32),
    )(pooled, w1, b1, w2, b2)


def kernel(x, pe, edge_index, batch, atom_emb, pe_W1, pe_b1, pe_W2, pe_b2,
           pe_bn_g, pe_bn_b, in_W, in_b, gin_W1, gin_b1, gin_W2, gin_b2,
           bn_g, bn_b, ro_W1, ro_b1, ro_W2, ro_b2):
    f32 = jnp.float32
    x2d = x.astype(jnp.int32).reshape(N, 1)

    pad = E_PAD - E
    src_pad = (jnp.arange(pad, dtype=jnp.int32) * 977) % N
    dst_pad = N + (jnp.arange(pad, dtype=jnp.int32) % NS)
    srcr = jnp.concatenate([edge_index[0].astype(jnp.int32), src_pad]
                           ).reshape(ROWS_TOT, ELANES)
    dstr = jnp.concatenate([edge_index[1].astype(jnp.int32), dst_pad]
                           ).reshape(ROWS_TOT, ELANES)
    batchr = batch.astype(jnp.int32).reshape(N // 125, 125)
    pool_zeros = jnp.zeros((G, HH), f32)

    r1 = lambda v: v.astype(f32).reshape(1, -1)

    xp, pe_st = _pe_pass(pe.astype(f32), pe_W1.astype(f32), r1(pe_b1),
                         pe_W2.astype(f32), r1(pe_b2))
    h = _h0_pass(x2d, xp, pe_st, atom_emb.astype(f32),
                 in_W[:A].astype(f32), in_W[A:].astype(f32), r1(in_b),
                 r1(pe_bn_g), r1(pe_bn_b))

    for l in range(L):
        z = _agg_kernel()(h, srcr, dstr)
        z2, st = _mlp_pass(z, gin_W1[l].astype(f32), r1(gin_b1[l]),
                           gin_W2[l].astype(f32), r1(gin_b2[l]))
        h = _bnrelu_pass(z2, st, r1(bn_g[l]), r1(bn_b[l]))

    hp = h.reshape(NC, N // 125, 125, HH)
    pooled = _pool_kernel()(hp, batchr, pool_zeros)
    out = _ro_pass(pooled, ro_W1.astype(f32), r1(ro_b1),
                   ro_W2.astype(f32), r1(ro_b2))
    return out.reshape(G)


# speedup vs baseline: 7.2451x; 1.7394x over previous
import functools

import jax
import jax.numpy as jnp
from jax import lax
from jax.experimental import pallas as pl
from jax.experimental.pallas import tpu as pltpu
from jax.experimental.pallas import tpu_sc as plsc

N = 50000
E = 800000
H = 64
HH = 32
A = 64
P = 16
PR = 20
NA = 28
G = 2000
L = 4

NC = 2
NS = 16

ELANES = 128
SUBCH = 16
NBUF = 4
LAG = 2
OUTER_ROWS = SUBCH
OUTER_N = 25
EPS_ROWS = OUTER_ROWS * OUTER_N
ROWS_TOT = EPS_ROWS * NS
E_PAD = ROWS_TOT * ELANES
CPY = 1000
NCPY = N // CPY

BLK = 2000
GRID = N // BLK

@functools.cache
def _mesh():
    return plsc.VectorSubcoreMesh(core_axis_name="c", subcore_axis_name="s",
                                  num_cores=NC, num_subcores=NS)


def _agg_body(h_hbm, srcr_hbm, dstr_hbm, z_hbm, acc, idx_s, idx_d,
              rows_a, rows_b, rows_c, rows_d, sem_g, sem_s):
    c = lax.axis_index("c")
    s = lax.axis_index("s")
    for k in range(4):
        cid = s + NS * k

        @pl.when(cid < NCPY)
        def _():
            pltpu.sync_copy(h_hbm.at[c, pl.ds(cid * CPY, CPY)],
                            acc.at[pl.ds(cid * CPY, CPY)])
    plsc.subcore_barrier()
    row0 = s * EPS_ROWS
    rows = [rows_a, rows_b, rows_c, rows_d]

    def outer(k, carry):
        r0 = row0 + k * OUTER_ROWS
        pltpu.sync_copy(srcr_hbm.at[pl.ds(r0, OUTER_ROWS)], idx_s)
        pltpu.sync_copy(dstr_hbm.at[pl.ds(r0, OUTER_ROWS)], idx_d)
        gd, sd = {}, {}
        for t in range(SUBCH + LAG):
            if t < SUBCH:
                if t >= NBUF:
                    sd[t - NBUF].wait()
                gd[t] = pltpu.async_copy(
                    h_hbm.at[c].at[idx_s.at[t]], rows[t % NBUF], sem_g)
            u = t - LAG
            if u >= 0:
                gd[u].wait()
                sd[u] = pltpu.async_copy(
                    rows[u % NBUF], acc.at[idx_d.at[u]], sem_s, add=True)
        for u in range(SUBCH - NBUF, SUBCH):
            sd[u].wait()
        return carry

    lax.fori_loop(0, OUTER_N, outer, 0)
    plsc.subcore_barrier()
    for k in range(4):
        cid = s + NS * k

        @pl.when(cid < NCPY)
        def _():
            pltpu.sync_copy(acc.at[pl.ds(cid * CPY, CPY)],
                            z_hbm.at[c, pl.ds(cid * CPY, CPY)])


@functools.cache
def _agg_kernel():
    return pl.kernel(
        _agg_body,
        out_type=jax.ShapeDtypeStruct((NC, N, HH), jnp.float32),
        mesh=_mesh(),
        compiler_params=pltpu.CompilerParams(use_tc_tiling_on_sc=False),
        scratch_types=[
            pltpu.VMEM_SHARED((N + NS, HH), jnp.float32),
            pltpu.VMEM((OUTER_ROWS, ELANES), jnp.int32),
            pltpu.VMEM((OUTER_ROWS, ELANES), jnp.int32),
            pltpu.VMEM((ELANES, HH), jnp.float32),
            pltpu.VMEM((ELANES, HH), jnp.float32),
            pltpu.VMEM((ELANES, HH), jnp.float32),
            pltpu.VMEM((ELANES, HH), jnp.float32),
            pltpu.SemaphoreType.DMA,
            pltpu.SemaphoreType.DMA,
        ],
    )


_PCHUNK = 8
_PNCH = (N // 125) // _PCHUNK
_GCPY = 200


def _pool_body(hp_hbm, batchr_hbm, zeros_hbm, out_hbm, acc, idx_b, rows):
    c = lax.axis_index("c")
    s = lax.axis_index("s")

    @pl.when(s < G // _GCPY)
    def _():
        pltpu.sync_copy(zeros_hbm.at[pl.ds(s * _GCPY, _GCPY)],
                        acc.at[pl.ds(s * _GCPY, _GCPY)])
    plsc.subcore_barrier()

    for k in range(4):
        cid = s + NS * k

        @pl.when(cid < _PNCH)
        def _():
            pltpu.sync_copy(batchr_hbm.at[pl.ds(cid * _PCHUNK, _PCHUNK)], idx_b)
            pltpu.sync_copy(hp_hbm.at[c, pl.ds(cid * _PCHUNK, _PCHUNK)], rows)
            for j in range(_PCHUNK):
                pltpu.sync_copy(rows.at[j], acc.at[idx_b.at[j]], add=True)

    plsc.subcore_barrier()

    @pl.when(s < G // _GCPY)
    def _():
        pltpu.sync_copy(acc.at[pl.ds(s * _GCPY, _GCPY)],
                        out_hbm.at[c, pl.ds(s * _GCPY, _GCPY)])


@functools.cache
def _pool_kernel():
    return pl.kernel(
        _pool_body,
        out_type=jax.ShapeDtypeStruct((NC, G, HH), jnp.float32),
        mesh=_mesh(),
        compiler_params=pltpu.CompilerParams(use_tc_tiling_on_sc=False),
        scratch_types=[
            pltpu.VMEM_SHARED((G, HH), jnp.float32),
            pltpu.VMEM((_PCHUNK, 125), jnp.int32),
            pltpu.VMEM((_PCHUNK, 125, HH), jnp.float32),
        ],
    )


def _pe_body(pe_ref, w1_ref, b1_ref, w2_ref, b2_ref, xp_ref, st_ref):
    i = pl.program_id(0)
    y = jnp.maximum(
        jnp.dot(pe_ref[...], w1_ref[...], preferred_element_type=jnp.float32,
                precision=lax.Precision.HIGHEST)
        + b1_ref[...], 0.0)
    z = jnp.dot(y, w2_ref[...], preferred_element_type=jnp.float32,
                precision=lax.Precision.HIGHEST) + b2_ref[...]
    xp_ref[...] = z

    @pl.when(i == 0)
    def _():
        st_ref[...] = jnp.zeros_like(st_ref)

    st_ref[0:1, :] += jnp.sum(z, axis=0, keepdims=True)
    st_ref[1:2, :] += jnp.sum(z * z, axis=0, keepdims=True)


def _pe_pass(pe, w1, b1, w2, b2):
    return pl.pallas_call(
        _pe_body,
        grid=(GRID,),
        in_specs=[
            pl.BlockSpec((BLK, PR), lambda i: (i, 0)),
            pl.BlockSpec((PR, P), lambda i: (0, 0)),
            pl.BlockSpec((1, P), lambda i: (0, 0)),
            pl.BlockSpec((P, P), lambda i: (0, 0)),
            pl.BlockSpec((1, P), lambda i: (0, 0)),
        ],
        out_specs=[
            pl.BlockSpec((BLK, P), lambda i: (i, 0)),
            pl.BlockSpec((2, P), lambda i: (0, 0)),
        ],
        out_shape=[
            jax.ShapeDtypeStruct((N, P), jnp.float32),
            jax.ShapeDtypeStruct((2, P), jnp.float32),
        ],
    )(pe, w1, b1, w2, b2)


def _h0_body(x_ref, xp_ref, st_ref, aemb_ref, wa_ref, wp_ref, inb_ref,
             peg_ref, peb_ref, out_ref):
    mu = st_ref[0:1, :] * (1.0 / N)
    var = st_ref[1:2, :] * (1.0 / N) - mu * mu
    scale = peg_ref[...] * lax.rsqrt(var + 1e-5)
    shift = peb_ref[...] - mu * scale
    onehot = (x_ref[...] == lax.broadcasted_iota(jnp.int32, (1, NA), 1)
              ).astype(jnp.float32)
    m = jnp.dot(aemb_ref[...], wa_ref[...], preferred_element_type=jnp.float32,
                precision=lax.Precision.HIGHEST)
    xa_part = jnp.dot(onehot, m, preferred_element_type=jnp.float32,
                precision=lax.Precision.HIGHEST)
    xp_bn = xp_ref[...] * scale + shift
    h0 = (xa_part
          + jnp.dot(xp_bn, wp_ref[...], preferred_element_type=jnp.float32,
                precision=lax.Precision.HIGHEST)
          + inb_ref[...])
    out_ref[0] = h0[:, :HH]
    out_ref[1] = h0[:, HH:]


def _h0_pass(x2d, xp, st, aemb, wa, wp, inb, peg, peb):
    return pl.pallas_call(
        _h0_body,
        grid=(GRID,),
        in_specs=[
            pl.BlockSpec((BLK, 1), lambda i: (i, 0)),
            pl.BlockSpec((BLK, P), lambda i: (i, 0)),
            pl.BlockSpec((2, P), lambda i: (0, 0)),
            pl.BlockSpec((NA, A), lambda i: (0, 0)),
            pl.BlockSpec((A, H), lambda i: (0, 0)),
            pl.BlockSpec((P, H), lambda i: (0, 0)),
            pl.BlockSpec((1, H), lambda i: (0, 0)),
            pl.BlockSpec((1, P), lambda i: (0, 0)),
            pl.BlockSpec((1, P), lambda i: (0, 0)),
        ],
        out_specs=pl.BlockSpec((NC, BLK, HH), lambda i: (0, i, 0)),
        out_shape=jax.ShapeDtypeStruct((NC, N, HH), jnp.float32),
    )(x2d, xp, st, aemb, wa, wp, inb, peg, peb)


def _mlp_body(z_ref, w1_ref, b1_ref, w2_ref, b2_ref, z2_ref, st_ref):
    i = pl.program_id(0)
    z = jnp.concatenate([z_ref[0], z_ref[1]], axis=1)
    y = jnp.maximum(
        jnp.dot(z, w1_ref[...], preferred_element_type=jnp.float32,
                precision=lax.Precision.HIGHEST)
        + b1_ref[...], 0.0)
    z2 = jnp.dot(y, w2_ref[...], preferred_element_type=jnp.float32,
                precision=lax.Precision.HIGHEST) + b2_ref[...]
    z2_ref[...] = z2

    @pl.when(i == 0)
    def _():
        st_ref[...] = jnp.zeros_like(st_ref)

    st_ref[0:1, :] += jnp.sum(z2, axis=0, keepdims=True)
    st_ref[1:2, :] += jnp.sum(z2 * z2, axis=0, keepdims=True)


def _mlp_pass(z_halves, w1, b1, w2, b2):
    return pl.pallas_call(
        _mlp_body,
        grid=(GRID,),
        in_specs=[
            pl.BlockSpec((NC, BLK, HH), lambda i: (0, i, 0)),
            pl.BlockSpec((H, H), lambda i: (0, 0)),
            pl.BlockSpec((1, H), lambda i: (0, 0)),
            pl.BlockSpec((H, H), lambda i: (0, 0)),
            pl.BlockSpec((1, H), lambda i: (0, 0)),
        ],
        out_specs=[
            pl.BlockSpec((BLK, H), lambda i: (i, 0)),
            pl.BlockSpec((2, H), lambda i: (0, 0)),
        ],
        out_shape=[
            jax.ShapeDtypeStruct((N, H), jnp.float32),
            jax.ShapeDtypeStruct((2, H), jnp.float32),
        ],
    )(z_halves, w1, b1, w2, b2)


def _bnrelu_body(z2_ref, st_ref, g_ref, b_ref, out_ref):
    mu = st_ref[0:1, :] * (1.0 / N)
    var = st_ref[1:2, :] * (1.0 / N) - mu * mu
    scale = g_ref[...] * lax.rsqrt(var + 1e-5)
    shift = b_ref[...] - mu * scale
    hh = jnp.maximum(z2_ref[...] * scale + shift, 0.0)
    out_ref[0] = hh[:, :HH]
    out_ref[1] = hh[:, HH:]


def _bnrelu_pass(z2, st, g, b):
    return pl.pallas_call(
        _bnrelu_body,
        grid=(GRID,),
        in_specs=[
            pl.BlockSpec((BLK, H), lambda i: (i, 0)),
            pl.BlockSpec((2, H), lambda i: (0, 0)),
            pl.BlockSpec((1, H), lambda i: (0, 0)),
            pl.BlockSpec((1, H), lambda i: (0, 0)),
        ],
        out_specs=pl.BlockSpec((NC, BLK, HH), lambda i: (0, i, 0)),
        out_shape=jax.ShapeDtypeStruct((NC, N, HH), jnp.float32),
    )(z2, st, g, b)


def _ro_body(p_ref, w1_ref, b1_ref, w2_ref, b2_ref, out_ref):
    p = jnp.concatenate([p_ref[0], p_ref[1]], axis=1)
    r = jnp.maximum(
        jnp.dot(p, w1_ref[...], preferred_element_type=jnp.float32,
                precision=lax.Precision.HIGHEST)
        + b1_ref[...], 0.0)
    out_ref[...] = (jnp.dot(r, w2_ref[...], preferred_element_type=jnp.float32,
                precision=lax.Precision.HIGHEST)
                    + b2_ref[...])


def _ro_pass(pooled, w1, b1, w2, b2):
    return pl.pallas_call(
        _ro_body,
        in_specs=[
            pl.BlockSpec((NC, G, HH), lambda: (0, 0, 0)),
            pl.BlockSpec((H, H), lambda: (0, 0)),
            pl.BlockSpec((1, H), lambda: (0, 0)),
            pl.BlockSpec((H, 1), lambda: (0, 0)),
            pl.BlockSpec((1, 1), lambda: (0, 0)),
        ],
        out_specs=pl.BlockSpec((G, 1), lambda: (0, 0)),
        out_shape=jax.ShapeDtypeStruct((G, 1), jnp.float32),
    )(pooled, w1, b1, w2, b2)


def kernel(x, pe, edge_index, batch, atom_emb, pe_W1, pe_b1, pe_W2, pe_b2,
           pe_bn_g, pe_bn_b, in_W, in_b, gin_W1, gin_b1, gin_W2, gin_b2,
           bn_g, bn_b, ro_W1, ro_b1, ro_W2, ro_b2):
    f32 = jnp.float32
    x2d = x.astype(jnp.int32).reshape(N, 1)

    pad = E_PAD - E
    src_pad = (jnp.arange(pad, dtype=jnp.int32) * 977) % N
    dst_pad = N + (jnp.arange(pad, dtype=jnp.int32) % NS)
    srcr = jnp.concatenate([edge_index[0].astype(jnp.int32), src_pad]
                           ).reshape(ROWS_TOT, ELANES)
    dstr = jnp.concatenate([edge_index[1].astype(jnp.int32), dst_pad]
                           ).reshape(ROWS_TOT, ELANES)
    batchr = batch.astype(jnp.int32).reshape(N // 125, 125)
    pool_zeros = jnp.zeros((G, HH), f32)

    r1 = lambda v: v.astype(f32).reshape(1, -1)

    xp, pe_st = _pe_pass(pe.astype(f32), pe_W1.astype(f32), r1(pe_b1),
                         pe_W2.astype(f32), r1(pe_b2))
    h = _h0_pass(x2d, xp, pe_st, atom_emb.astype(f32),
                 in_W[:A].astype(f32), in_W[A:].astype(f32), r1(in_b),
                 r1(pe_bn_g), r1(pe_bn_b))

    for l in range(L):
        z = _agg_kernel()(h, srcr, dstr)
        z2, st = _mlp_pass(z, gin_W1[l].astype(f32), r1(gin_b1[l]),
                           gin_W2[l].astype(f32), r1(gin_b2[l]))
        h = _bnrelu_pass(z2, st, r1(bn_g[l]), r1(bn_b[l]))

    hp = h.reshape(NC, N // 125, 125, HH)
    pooled = _pool_kernel()(hp, batchr, pool_zeros)
    out = _ro_pass(pooled, ro_W1.astype(f32), r1(ro_b1),
                   ro_W2.astype(f32), r1(ro_b2))
    return out.reshape(G)
